# baseline (device time: 42712 ns/iter reference)
import jax
import jax.numpy as jnp
from jax import lax
from jax.experimental import pallas as pl
from jax.experimental.pallas import tpu as pltpu

N_DEV = 8
SQ = 256
SKV = 4096
H = 8
DH = 128
D = 1024
SQH = SQ // 2
SCALE = 0.08838834764831843

MASKS = (4, 1, 3)


def kernel(x, Wq, Wo, K_ext, V_ext):
    x2 = x.reshape(SQ, 1024)
    K3 = K_ext.reshape(SKV, H, DH)
    V3 = V_ext.reshape(SKV, H, DH)

    def body(x_ref, wq_ref, wo_ref, k_hbm, v_hbm, out_ref,
             kbuf, vbuf, acc_ref, sbuf_ref, rbuf_ref,
             kv_sems, send_sems, recv_sems):
        my_pos = lax.axis_index("i")

        barrier_sem = pltpu.get_barrier_semaphore()
        for mask in MASKS:
            pl.semaphore_signal(
                barrier_sem, inc=1,
                device_id=(jnp.bitwise_xor(my_pos, mask),),
                device_id_type=pl.DeviceIdType.MESH,
            )
        pl.semaphore_wait(barrier_sem, len(MASKS))

        def kv_dma(h, slot):
            half = SKV // 2
            return (
                pltpu.make_async_copy(
                    k_hbm.at[0:half, h, :], kbuf.at[slot, 0:half],
                    kv_sems.at[slot, 0]),
                pltpu.make_async_copy(
                    k_hbm.at[half:SKV, h, :], kbuf.at[slot, half:SKV],
                    kv_sems.at[slot, 1]),
                pltpu.make_async_copy(
                    v_hbm.at[0:half, h, :], vbuf.at[slot, 0:half],
                    kv_sems.at[slot, 2]),
                pltpu.make_async_copy(
                    v_hbm.at[half:SKV, h, :], vbuf.at[slot, half:SKV],
                    kv_sems.at[slot, 3]),
            )

        for d in kv_dma(0, 0):
            d.start()

        xb = x_ref[...].astype(jnp.bfloat16)
        wqb = wq_ref[...].astype(jnp.bfloat16)
        wob = wo_ref[...].astype(jnp.bfloat16)

        acc = jnp.zeros((SQ, D), jnp.float32)
        for h in range(H):
            slot = h % 2
            if h + 1 < H:
                for d in kv_dma(h + 1, 1 - slot):
                    d.start()
            q = jnp.dot(xb, wqb[:, h * DH:(h + 1) * DH],
                        preferred_element_type=jnp.float32)
            q = q * (SCALE * 1.4426950408889634)
            for d in kv_dma(h, slot):
                d.wait()
            k = kbuf[slot].astype(jnp.bfloat16)
            v = vbuf[slot].astype(jnp.bfloat16)
            s = lax.dot_general(
                q.astype(jnp.bfloat16), k, (((1,), (1,)), ((), ())),
                preferred_element_type=jnp.float32,
            )
            p = jnp.exp2(s)
            l = jnp.sum(p, axis=1, keepdims=True)
            o = jnp.dot(p.astype(jnp.bfloat16), v,
                        preferred_element_type=jnp.float32)
            att = (o / l).astype(jnp.bfloat16)
            acc = acc + jnp.dot(att, wob[h * DH:(h + 1) * DH, :],
                                preferred_element_type=jnp.float32)

        acc_h = [acc[:SQH, :], acc[SQH:, :]]
        orders = (MASKS, (MASKS[1], MASKS[2], MASKS[0]))
        for ph in range(3):
            rdmas = []
            for half in range(2):
                mask = orders[half][ph]
                sbuf_ref[half] = acc_h[half].astype(jnp.bfloat16)
                rdmas.append(pltpu.make_async_remote_copy(
                    src_ref=sbuf_ref.at[half],
                    dst_ref=rbuf_ref.at[ph, half],
                    send_sem=send_sems.at[ph, half],
                    recv_sem=recv_sems.at[ph, half],
                    device_id=(jnp.bitwise_xor(my_pos, mask),),
                    device_id_type=pl.DeviceIdType.MESH,
                ))
            for r in rdmas:
                r.start()
            for half in range(2):
                rdmas[half].wait()
                acc_h[half] = acc_h[half] + rbuf_ref[ph, half].astype(
                    jnp.float32)

        out_ref[:SQH, :] = acc_h[0]
        out_ref[SQH:, :] = acc_h[1]
        del acc_ref

    out = pl.pallas_call(
        body,
        out_shape=jax.ShapeDtypeStruct((SQ, D), jnp.float32),
        in_specs=[
            pl.BlockSpec(memory_space=pltpu.VMEM),
            pl.BlockSpec(memory_space=pltpu.VMEM),
            pl.BlockSpec(memory_space=pltpu.VMEM),
            pl.BlockSpec(memory_space=pltpu.MemorySpace.HBM),
            pl.BlockSpec(memory_space=pltpu.MemorySpace.HBM),
        ],
        out_specs=pl.BlockSpec(memory_space=pltpu.VMEM),
        scratch_shapes=[
            pltpu.VMEM((2, SKV, DH), jnp.float32),
            pltpu.VMEM((2, SKV, DH), jnp.float32),
            pltpu.VMEM((SQ, D), jnp.float32),
            pltpu.VMEM((2, SQH, D), jnp.bfloat16),
            pltpu.VMEM((3, 2, SQH, D), jnp.bfloat16),
            pltpu.SemaphoreType.DMA((2, 4)),
            pltpu.SemaphoreType.DMA((3, 2)),
            pltpu.SemaphoreType.DMA((3, 2)),
        ],
        compiler_params=pltpu.CompilerParams(
            collective_id=0,
            vmem_limit_bytes=100 * 1024 * 1024,
        ),
    )(x2, Wq, Wo, K3, V3)
    return out.reshape(1, SQ, D)


# device time: 42344 ns/iter; 1.0087x vs baseline; 1.0087x over previous
import jax
import jax.numpy as jnp
from jax import lax
from jax.experimental import pallas as pl
from jax.experimental.pallas import tpu as pltpu

N_DEV = 8
SQ = 256
SKV = 4096
H = 8
DH = 128
D = 1024
SQH = SQ // 2
SCALE = 0.08838834764831843

MASKS = (4, 1, 3)


def kernel(x, Wq, Wo, K_ext, V_ext):
    x2 = x.reshape(SQ, 1024)
    K3 = K_ext.reshape(SKV, H, DH)
    V3 = V_ext.reshape(SKV, H, DH)

    def body(x_ref, wq_ref, wo_ref, k_hbm, v_hbm, out_ref,
             kbuf, vbuf, acc_ref, sbuf_ref, rbuf_ref,
             kv_sems, send_sems, recv_sems):
        my_pos = lax.axis_index("i")

        barrier_sem = pltpu.get_barrier_semaphore()
        for mask in MASKS:
            pl.semaphore_signal(
                barrier_sem, inc=1,
                device_id=(jnp.bitwise_xor(my_pos, mask),),
                device_id_type=pl.DeviceIdType.MESH,
            )

        def kv_dma(h, slot):
            half = SKV // 2
            return (
                pltpu.make_async_copy(
                    k_hbm.at[0:half, h, :], kbuf.at[slot, 0:half],
                    kv_sems.at[slot, 0]),
                pltpu.make_async_copy(
                    k_hbm.at[half:SKV, h, :], kbuf.at[slot, half:SKV],
                    kv_sems.at[slot, 1]),
                pltpu.make_async_copy(
                    v_hbm.at[0:half, h, :], vbuf.at[slot, 0:half],
                    kv_sems.at[slot, 2]),
                pltpu.make_async_copy(
                    v_hbm.at[half:SKV, h, :], vbuf.at[slot, half:SKV],
                    kv_sems.at[slot, 3]),
            )

        for d in kv_dma(0, 0):
            d.start()

        xb = x_ref[...].astype(jnp.bfloat16)
        wqb = wq_ref[...].astype(jnp.bfloat16)
        wob = wo_ref[...].astype(jnp.bfloat16)

        acc = jnp.zeros((SQ, D), jnp.float32)
        for h in range(H):
            slot = h % 2
            if h + 1 < H:
                for d in kv_dma(h + 1, 1 - slot):
                    d.start()
            q = jnp.dot(xb, wqb[:, h * DH:(h + 1) * DH],
                        preferred_element_type=jnp.float32)
            q = q * (SCALE * 1.4426950408889634)
            for d in kv_dma(h, slot):
                d.wait()
            k = kbuf[slot].astype(jnp.bfloat16)
            v = vbuf[slot].astype(jnp.bfloat16)
            s = lax.dot_general(
                q.astype(jnp.bfloat16), k, (((1,), (1,)), ((), ())),
                preferred_element_type=jnp.float32,
            )
            p = jnp.exp2(s)
            l = jnp.sum(p, axis=1, keepdims=True)
            o = jnp.dot(p.astype(jnp.bfloat16), v,
                        preferred_element_type=jnp.float32)
            att = (o / l).astype(jnp.bfloat16)
            acc = acc + jnp.dot(att, wob[h * DH:(h + 1) * DH, :],
                                preferred_element_type=jnp.float32)

        pl.semaphore_wait(barrier_sem, len(MASKS))

        acc_h = [acc[:SQH, :], acc[SQH:, :]]
        orders = (MASKS, (MASKS[1], MASKS[2], MASKS[0]))
        for ph in range(3):
            rdmas = []
            for half in range(2):
                mask = orders[half][ph]
                sbuf_ref[half] = acc_h[half].astype(jnp.bfloat16)
                rdmas.append(pltpu.make_async_remote_copy(
                    src_ref=sbuf_ref.at[half],
                    dst_ref=rbuf_ref.at[ph, half],
                    send_sem=send_sems.at[ph, half],
                    recv_sem=recv_sems.at[ph, half],
                    device_id=(jnp.bitwise_xor(my_pos, mask),),
                    device_id_type=pl.DeviceIdType.MESH,
                ))
            for r in rdmas:
                r.start()
            for half in range(2):
                rdmas[half].wait()
                acc_h[half] = acc_h[half] + rbuf_ref[ph, half].astype(
                    jnp.float32)

        out_ref[:SQH, :] = acc_h[0]
        out_ref[SQH:, :] = acc_h[1]
        del acc_ref

    out = pl.pallas_call(
        body,
        out_shape=jax.ShapeDtypeStruct((SQ, D), jnp.float32),
        in_specs=[
            pl.BlockSpec(memory_space=pltpu.VMEM),
            pl.BlockSpec(memory_space=pltpu.VMEM),
            pl.BlockSpec(memory_space=pltpu.VMEM),
            pl.BlockSpec(memory_space=pltpu.MemorySpace.HBM),
            pl.BlockSpec(memory_space=pltpu.MemorySpace.HBM),
        ],
        out_specs=pl.BlockSpec(memory_space=pltpu.VMEM),
        scratch_shapes=[
            pltpu.VMEM((2, SKV, DH), jnp.float32),
            pltpu.VMEM((2, SKV, DH), jnp.float32),
            pltpu.VMEM((SQ, D), jnp.float32),
            pltpu.VMEM((2, SQH, D), jnp.bfloat16),
            pltpu.VMEM((3, 2, SQH, D), jnp.bfloat16),
            pltpu.SemaphoreType.DMA((2, 4)),
            pltpu.SemaphoreType.DMA((3, 2)),
            pltpu.SemaphoreType.DMA((3, 2)),
        ],
        compiler_params=pltpu.CompilerParams(
            collective_id=0,
            vmem_limit_bytes=100 * 1024 * 1024,
        ),
    )(x2, Wq, Wo, K3, V3)
    return out.reshape(1, SQ, D)
